# baseline (speedup 1.0000x reference)
import functools

import jax
import jax.numpy as jnp
from jax import lax
from jax.experimental import pallas as pl
from jax.experimental.pallas import tpu as pltpu
from jax.experimental.pallas import tpu_sc as plsc

_C = 80


@functools.lru_cache(maxsize=None)
def _make_sc_partial(N, D, E):
    info = plsc.get_sparse_core_info()
    NC, NS, L = info.num_cores, info.num_subcores, info.num_lanes
    NW = NC * NS
    assert E % (NW * _C) == 0 and D % L == 0
    EW = E // NW
    NCH = EW // _C
    KD = D // L
    assert NCH % 2 == 1
    NP = ((N + NS * _C - 1) // (NS * _C)) * (NS * _C)
    RPT = NP // NS
    RB = RPT // _C

    mesh = plsc.VectorSubcoreMesh(core_axis_name="c", subcore_axis_name="s")

    @functools.partial(
        pl.kernel,
        out_type=jax.ShapeDtypeStruct((NC, NP, D), jnp.float32),
        mesh=mesh,
        scratch_types=[
            pltpu.VMEM((EW,), jnp.int32),
            pltpu.VMEM((EW,), jnp.int32),
            pltpu.VMEM((2, _C + 16), jnp.float32),
            pltpu.VMEM((2, _C), jnp.int32),
            pltpu.VMEM((2, _C, D), jnp.float32),
            pltpu.VMEM_SHARED((NP, D), jnp.float32),
            pltpu.SemaphoreType.DMA,
            pltpu.SemaphoreType.DMA,
            pltpu.SemaphoreType.DMA,
            pltpu.SemaphoreType.DMA,
            pltpu.SemaphoreType.DMA,
        ],
    )
    def sc_partial(row_hbm, col_hbm, val_hbm, ego_hbm, out_hbm,
                   rows_a, cols_a, vbuf, ridx, gbuf, acc,
                   sem_m, sem_g0, sem_g1, sem_s0, sem_s1):
        c = lax.axis_index("c")
        s = lax.axis_index("s")
        wid = c * NS + s
        base_r = s * RPT
        ebase = wid * EW
        sem_g = (sem_g0, sem_g1)
        sem_s = (sem_s0, sem_s1)

        m0 = pltpu.async_copy(row_hbm.at[pl.ds(ebase, EW)], rows_a, sem_m)
        m1 = pltpu.async_copy(col_hbm.at[pl.ds(ebase, EW)], cols_a, sem_m)

        @plsc.parallel_loop(0, _C)
        def _(e):
            for k in range(KD):
                gbuf[0, e, pl.ds(k * L, L)] = jnp.zeros((L,), jnp.float32)
        for i in range(RB):
            pltpu.sync_copy(gbuf.at[0], acc.at[pl.ds(base_r + i * _C, _C)])
        m0.wait()
        m1.wait()
        plsc.subcore_barrier()

        def start_gather(b, ch):
            pltpu.async_copy(val_hbm.at[pl.ds(ebase + ch * _C, _C)],
                             vbuf.at[b, pl.ds(0, _C)], sem_g[b])
            pltpu.async_copy(
                ego_hbm.at[cols_a.at[pl.ds(ch * _C, _C)]], gbuf.at[b], sem_g[b])

        def wait_gather(b):
            pltpu.make_async_copy(val_hbm.at[pl.ds(0, _C)],
                                  vbuf.at[b, pl.ds(0, _C)], sem_g[b]).wait()
            pltpu.make_async_copy(
                ego_hbm.at[cols_a.at[pl.ds(0, _C)]], gbuf.at[b], sem_g[b]).wait()

        def scale(b, ch):
            @plsc.parallel_loop(0, _C // L)
            def _(g):
                ridx[b, pl.ds(g * L, L)] = rows_a[pl.ds(ch * _C + g * L, L)]

            @plsc.parallel_loop(0, _C, unroll=8)
            def _(e):
                bv = jnp.full((L,), vbuf[b, pl.ds(e, L)][0], jnp.float32)
                for k in range(KD):
                    sl = pl.ds(k * L, L)
                    gbuf[b, e, sl] = gbuf[b, e, sl] * bv

        def start_scatter(b):
            pltpu.async_copy(gbuf.at[b], acc.at[ridx.at[b]], sem_s[b],
                             add=True)

        def wait_scatter(b):
            pltpu.make_async_copy(gbuf.at[b], acc.at[ridx.at[b]],
                                  sem_s[b]).wait()

        start_gather(0, 0)

        def pair(i, _):
            @pl.when(i > 0)
            def _():
                wait_scatter(1)
            start_gather(1, 2 * i + 1)
            wait_gather(0)
            scale(0, 2 * i)
            start_scatter(0)
            wait_gather(1)
            scale(1, 2 * i + 1)
            wait_scatter(0)
            start_gather(0, 2 * i + 2)
            start_scatter(1)
            return 0
        lax.fori_loop(0, (NCH - 1) // 2, pair, 0)

        wait_scatter(1)
        wait_gather(0)
        scale(0, NCH - 1)
        start_scatter(0)
        wait_scatter(0)
        plsc.subcore_barrier()

        for i in range(RB):
            pltpu.sync_copy(acc.at[pl.ds(base_r + i * _C, _C)],
                            out_hbm.at[c, pl.ds(base_r + i * _C, _C)])

    return sc_partial


@functools.lru_cache(maxsize=None)
def _make_combine(N, NP, D):
    BR = 1000
    assert N % BR == 0

    def body(p0_ref, p1_ref, o_ref):
        o_ref[...] = p0_ref[...] + p1_ref[...]

    return pl.pallas_call(
        body,
        grid=(N // BR,),
        in_specs=[pl.BlockSpec((BR, D), lambda i: (i, 0)),
                  pl.BlockSpec((BR, D), lambda i: (i, 0))],
        out_specs=pl.BlockSpec((BR, D), lambda i: (i, 0)),
        out_shape=jax.ShapeDtypeStruct((N, D), jnp.float32),
    )


def kernel(ego_embeddings, edge_index, edge_vals):
    N, D = ego_embeddings.shape
    E = edge_vals.shape[0]
    p = _make_sc_partial(N, D, E)(
        edge_index[0], edge_index[1], edge_vals, ego_embeddings)
    NP = p.shape[1]
    return _make_combine(N, NP, D)(p[0], p[1])


# speedup vs baseline: 1.1677x; 1.1677x over previous
import functools

import jax
import jax.numpy as jnp
from jax import lax
from jax.experimental import pallas as pl
from jax.experimental.pallas import tpu as pltpu
from jax.experimental.pallas import tpu_sc as plsc

_C = 80
_NB = 3


@functools.lru_cache(maxsize=None)
def _make_sc_partial(N, D, E):
    info = plsc.get_sparse_core_info()
    NC, NS, L = info.num_cores, info.num_subcores, info.num_lanes
    NW = NC * NS
    assert E % (NW * _C) == 0 and D % L == 0
    EW = E // NW
    NCH = EW // _C
    KD = D // L
    NCH_BODY = (NCH // _NB) * _NB if (NCH // _NB) * _NB <= NCH - 2 else NCH - 2
    assert NCH - NCH_BODY == 2
    NP = ((N + NS * _C - 1) // (NS * _C)) * (NS * _C)
    RPT = NP // NS
    RB = RPT // _C

    mesh = plsc.VectorSubcoreMesh(core_axis_name="c", subcore_axis_name="s")

    @functools.partial(
        pl.kernel,
        out_type=jax.ShapeDtypeStruct((NC, NP, D), jnp.float32),
        mesh=mesh,
        scratch_types=[
            pltpu.VMEM((EW,), jnp.int32),
            pltpu.VMEM((_NB, _C + 16), jnp.float32),
            pltpu.VMEM((_NB, _C), jnp.int32),
            pltpu.VMEM((_NB, _C, D), jnp.float32),
            pltpu.VMEM_SHARED((NP, D), jnp.float32),
            pltpu.SemaphoreType.DMA,
            pltpu.SemaphoreType.DMA,
            pltpu.SemaphoreType.DMA,
            pltpu.SemaphoreType.DMA,
            pltpu.SemaphoreType.DMA,
            pltpu.SemaphoreType.DMA,
            pltpu.SemaphoreType.DMA,
        ],
    )
    def sc_partial(row_hbm, col_hbm, val_hbm, ego_hbm, out_hbm,
                   cols_a, vbuf, ridx, gbuf, acc,
                   sem_m, sem_g0, sem_g1, sem_g2, sem_s0, sem_s1, sem_s2):
        c = lax.axis_index("c")
        s = lax.axis_index("s")
        wid = c * NS + s
        base_r = s * RPT
        ebase = wid * EW
        sem_g = (sem_g0, sem_g1, sem_g2)
        sem_s = (sem_s0, sem_s1, sem_s2)

        m1 = pltpu.async_copy(col_hbm.at[pl.ds(ebase, EW)], cols_a, sem_m)

        @plsc.parallel_loop(0, _C)
        def _(e):
            for k in range(KD):
                gbuf[0, e, pl.ds(k * L, L)] = jnp.zeros((L,), jnp.float32)
        for i in range(RB):
            pltpu.sync_copy(gbuf.at[0], acc.at[pl.ds(base_r + i * _C, _C)])
        m1.wait()
        plsc.subcore_barrier()

        def start_gather(b, ch):
            pltpu.async_copy(val_hbm.at[pl.ds(ebase + ch * _C, _C)],
                             vbuf.at[b, pl.ds(0, _C)], sem_g[b])
            pltpu.async_copy(row_hbm.at[pl.ds(ebase + ch * _C, _C)],
                             ridx.at[b], sem_g[b])
            pltpu.async_copy(
                ego_hbm.at[cols_a.at[pl.ds(ch * _C, _C)]], gbuf.at[b], sem_g[b])

        def wait_gather(b):
            pltpu.make_async_copy(val_hbm.at[pl.ds(0, _C)],
                                  vbuf.at[b, pl.ds(0, _C)], sem_g[b]).wait()
            pltpu.make_async_copy(row_hbm.at[pl.ds(0, _C)],
                                  ridx.at[b], sem_g[b]).wait()
            pltpu.make_async_copy(
                ego_hbm.at[cols_a.at[pl.ds(0, _C)]], gbuf.at[b], sem_g[b]).wait()

        def scale(b, ch):
            @plsc.parallel_loop(0, _C, unroll=8)
            def _(e):
                bv = jnp.full((L,), vbuf[b, pl.ds(e, L)][0], jnp.float32)
                for k in range(KD):
                    sl = pl.ds(k * L, L)
                    gbuf[b, e, sl] = gbuf[b, e, sl] * bv

        def start_scatter(b):
            pltpu.async_copy(gbuf.at[b], acc.at[ridx.at[b]], sem_s[b],
                             add=True)

        def wait_scatter(b):
            pltpu.make_async_copy(gbuf.at[b], acc.at[ridx.at[b]],
                                  sem_s[b]).wait()

        start_gather(0, 0)
        start_gather(1, 1)

        def body(i, _):
            for k in range(_NB):
                ch = _NB * i + k
                b = k
                bp = (k + 2) % _NB
                wait_gather(b)
                scale(b, ch)
                if k == 0:
                    @pl.when(i > 0)
                    def _():
                        wait_scatter(bp)
                else:
                    wait_scatter(bp)
                start_gather(bp, ch + 2)
                start_scatter(b)
            return 0
        lax.fori_loop(0, NCH_BODY // _NB, body, 0)

        for t in range(2):
            ch = NCH_BODY + t
            b = ch % _NB
            wait_gather(b)
            scale(b, ch)
            wait_scatter((b + 2) % _NB)
            start_scatter(b)
        wait_scatter((NCH - 1) % _NB)
        plsc.subcore_barrier()

        for i in range(RB):
            pltpu.sync_copy(acc.at[pl.ds(base_r + i * _C, _C)],
                            out_hbm.at[c, pl.ds(base_r + i * _C, _C)])

    return sc_partial


@functools.lru_cache(maxsize=None)
def _make_combine(N, NP, D):
    BR = 1000
    assert N % BR == 0

    def body(p0_ref, p1_ref, o_ref):
        o_ref[...] = p0_ref[...] + p1_ref[...]

    return pl.pallas_call(
        body,
        grid=(N // BR,),
        in_specs=[pl.BlockSpec((BR, D), lambda i: (i, 0)),
                  pl.BlockSpec((BR, D), lambda i: (i, 0))],
        out_specs=pl.BlockSpec((BR, D), lambda i: (i, 0)),
        out_shape=jax.ShapeDtypeStruct((N, D), jnp.float32),
    )


def kernel(ego_embeddings, edge_index, edge_vals):
    N, D = ego_embeddings.shape
    E = edge_vals.shape[0]
    p = _make_sc_partial(N, D, E)(
        edge_index[0], edge_index[1], edge_vals, ego_embeddings)
    NP = p.shape[1]
    return _make_combine(N, NP, D)(p[0], p[1])
